# baseline (speedup 1.0000x reference)
import jax
import jax.numpy as jnp
from jax import lax
from jax.experimental import pallas as pl
from jax.experimental.pallas import tpu as pltpu
from jax.experimental.pallas import tpu_sc as plsc

NUM_ROWS = 1_000_000
D = 64
NB = 16384
NJ = 26
B_TOTAL = NB * NJ
NW = 32
B_PER_W = B_TOTAL // NW
BW = NB // NW
CHUNK = 128
N_CHUNKS = B_PER_W // CHUNK
NBUF = 8
N_GROUPS = N_CHUNKS // NBUF


def _emb_kernel(zT_hbm, table_hbm, out_hbm, z_v, idx_v, rows, gsems, ssems,
                idx_sem):
    wid = lax.axis_index("s") * 2 + lax.axis_index("c")
    base = wid * B_PER_W
    b0 = wid * BW

    for j in range(NJ):
        pltpu.async_copy(zT_hbm.at[j, pl.ds(b0, BW)],
                         z_v.at[pl.ds(j * BW, BW)], idx_sem)
    for j in range(NJ):
        pltpu.make_async_copy(zT_hbm.at[j, pl.ds(b0, BW)],
                              z_v.at[pl.ds(j * BW, BW)], idx_sem).wait()

    lane = lax.iota(jnp.int32, 16)

    def build(t, carry):
        q0 = t * 16
        q = q0 + lane
        b_loc = (q * 80660) >> 21
        j = q - NJ * b_loc
        vals = plsc.load_gather(z_v, [j * BW + b_loc])
        idx_v[q0 >> 7, pl.ds(q0 & 127, 16)] = vals
        return carry

    lax.fori_loop(0, B_PER_W // 16, build, 0)

    def gather_refs(j, b):
        return (table_hbm.at[idx_v.at[j]], rows.at[b], gsems.at[b])

    def store_refs(j, b):
        return (rows.at[b], out_hbm.at[pl.ds(base + j * CHUNK, CHUNK)],
                ssems.at[b])

    for b in range(NBUF):
        pltpu.async_copy(*gather_refs(b, b))

    def group(g, carry):
        for b in range(NBUF):
            j = g * NBUF + b
            pltpu.make_async_copy(*gather_refs(j, b)).wait()
            pltpu.async_copy(*store_refs(j, b))

            @pl.when(g < N_GROUPS - 1)
            def _():
                pltpu.make_async_copy(*store_refs(j, b)).wait()
                pltpu.async_copy(*gather_refs(j + NBUF, b))
        return carry

    lax.fori_loop(0, N_GROUPS, group, 0)

    for b in range(NBUF):
        j = (N_GROUPS - 1) * NBUF + b
        pltpu.make_async_copy(*store_refs(j, b)).wait()


@jax.jit
def kernel(Z, weight):
    zT = Z.astype(jnp.int32).T
    mesh = plsc.VectorSubcoreMesh(core_axis_name="c", subcore_axis_name="s")
    out = pl.kernel(
        _emb_kernel,
        out_type=jax.ShapeDtypeStruct((B_TOTAL, D), jnp.float32),
        mesh=mesh,
        scratch_types=[
            pltpu.VMEM((NJ * BW,), jnp.int32),
            pltpu.VMEM((N_CHUNKS, CHUNK), jnp.int32),
            pltpu.VMEM((NBUF, CHUNK, D), jnp.float32),
            pltpu.SemaphoreType.DMA((NBUF,)),
            pltpu.SemaphoreType.DMA((NBUF,)),
            pltpu.SemaphoreType.DMA,
        ],
        compiler_params=pltpu.CompilerParams(
            use_tc_tiling_on_sc=False, needs_layout_passes=False),
    )(zT, weight)
    return out.reshape(NB, NJ, D)


# speedup vs baseline: 1.0036x; 1.0036x over previous
import jax
import jax.numpy as jnp
from jax import lax
from jax.experimental import pallas as pl
from jax.experimental.pallas import tpu as pltpu
from jax.experimental.pallas import tpu_sc as plsc

NUM_ROWS = 1_000_000
D = 64
NB = 16384
NJ = 26
B_TOTAL = NB * NJ
NW = 32
B_PER_W = B_TOTAL // NW
CHUNK = 128
N_CHUNKS = B_PER_W // CHUNK
NBUF = 8
N_GROUPS = N_CHUNKS // NBUF


def _emb_kernel(idx_hbm, table_hbm, out_hbm, idx_v, rows, gsems, ssems,
                idx_sem):
    wid = lax.axis_index("s") * 2 + lax.axis_index("c")
    base = wid * B_PER_W

    pltpu.async_copy(idx_hbm.at[pl.ds(base, B_PER_W)], idx_v, idx_sem).wait()

    def gather_refs(j, b):
        return (table_hbm.at[idx_v.at[pl.ds(j * CHUNK, CHUNK)]], rows.at[b],
                gsems.at[b])

    def store_refs(j, b):
        return (rows.at[b], out_hbm.at[pl.ds(base + j * CHUNK, CHUNK)],
                ssems.at[b])

    for b in range(NBUF):
        pltpu.async_copy(*gather_refs(b, b))

    def group(g, carry):
        for b in range(NBUF):
            j = g * NBUF + b
            pltpu.make_async_copy(*gather_refs(j, b)).wait()
            pltpu.async_copy(*store_refs(j, b))

            @pl.when(g < N_GROUPS - 1)
            def _():
                pltpu.make_async_copy(*store_refs(j, b)).wait()
                pltpu.async_copy(*gather_refs(j + NBUF, b))
        return carry

    lax.fori_loop(0, N_GROUPS, group, 0)

    for b in range(NBUF):
        j = (N_GROUPS - 1) * NBUF + b
        pltpu.make_async_copy(*store_refs(j, b)).wait()


@jax.jit
def kernel(Z, weight):
    idx = Z.astype(jnp.int32).reshape(B_TOTAL)
    mesh = plsc.VectorSubcoreMesh(core_axis_name="c", subcore_axis_name="s")
    out = pl.kernel(
        _emb_kernel,
        out_type=jax.ShapeDtypeStruct((B_TOTAL, D), jnp.float32),
        mesh=mesh,
        scratch_types=[
            pltpu.VMEM((B_PER_W,), jnp.int32),
            pltpu.VMEM((NBUF, CHUNK, D), jnp.float32),
            pltpu.SemaphoreType.DMA((NBUF,)),
            pltpu.SemaphoreType.DMA((NBUF,)),
            pltpu.SemaphoreType.DMA,
        ],
        compiler_params=pltpu.CompilerParams(
            use_tc_tiling_on_sc=False, needs_layout_passes=False),
    )(idx, weight)
    return out.reshape(NB, NJ, D)
